# baseline (speedup 1.0000x reference)
import functools

import jax
import jax.numpy as jnp
from jax import lax
from jax.experimental import pallas as pl
from jax.experimental.pallas import tpu as pltpu
from jax.experimental.pallas import tpu_sc as plsc

BATCH = 16384
SEQ = 50
D_MODEL = 64
NUM_WORKERS = 32
ROWS_PER_W = BATCH // NUM_WORKERS
K = 1
NBUF = 8
NCHUNK = ROWS_PER_W // K
NGROUPS = NCHUNK // NBUF


def _make_kernel():
    mesh = plsc.VectorSubcoreMesh(core_axis_name="c", subcore_axis_name="s")

    @functools.partial(
        pl.kernel,
        mesh=mesh,
        out_type=jax.ShapeDtypeStruct((BATCH, SEQ, D_MODEL), jnp.float32),
        scratch_types=[
            pltpu.VMEM((ROWS_PER_W, SEQ), jnp.int32),
            pltpu.VMEM((NBUF, SEQ, D_MODEL), jnp.float32),
            pltpu.SemaphoreType.DMA,
            pltpu.SemaphoreType.DMA,
        ],
        compiler_params=pltpu.CompilerParams(use_tc_tiling_on_sc=False),
    )
    def k(table_hbm, x_hbm, out_hbm, idx_v, bufs, gsem, wsem):
        num_cores = 2
        wid = lax.axis_index("s") * num_cores + lax.axis_index("c")
        row0 = wid * ROWS_PER_W
        pltpu.sync_copy(x_hbm.at[pl.ds(row0, ROWS_PER_W)], idx_v)

        def gather(j, b):
            return pltpu.make_async_copy(
                table_hbm.at[idx_v.at[j]], bufs.at[b], gsem)

        def wback(j, b):
            return pltpu.make_async_copy(
                bufs.at[b], out_hbm.at[row0 + j], wsem)

        for b in range(NBUF):
            gather(b, b).start()

        def group(g, carry):
            g0 = g * NBUF
            for b in range(NBUF):
                gather(g0 + b, b).wait()
                wback(g0 + b, b).start()
            for b in range(NBUF):
                wback(g0 + b, b).wait()
                gather(g0 + NBUF + b, b).start()
            return carry

        lax.fori_loop(0, NGROUPS - 1, group, 0)

        g0 = (NGROUPS - 1) * NBUF
        for b in range(NBUF):
            gather(g0 + b, b).wait()
            wback(g0 + b, b).start()
        for b in range(NBUF):
            wback(g0 + b, b).wait()

    return k


_gather_kernel = _make_kernel()


@jax.jit
def kernel(x, table):
    return _gather_kernel(table, x)


# speedup vs baseline: 1.0148x; 1.0148x over previous
import functools

import jax
import jax.numpy as jnp
from jax import lax
from jax.experimental import pallas as pl
from jax.experimental.pallas import tpu as pltpu
from jax.experimental.pallas import tpu_sc as plsc

B_TOTAL = 16384 * 50
D_MODEL = 64
NUM_WORKERS = 32
PER_WORKER = B_TOTAL // NUM_WORKERS
CHUNK = 200
NCHUNK = PER_WORKER // CHUNK
NGRID = B_TOTAL // CHUNK
NBUF = 8
NGROUPS = NCHUNK // NBUF


def _make_kernel():
    mesh = plsc.VectorSubcoreMesh(core_axis_name="c", subcore_axis_name="s")

    @functools.partial(
        pl.kernel,
        mesh=mesh,
        out_type=jax.ShapeDtypeStruct((NGRID, CHUNK, D_MODEL), jnp.float32),
        scratch_types=[
            pltpu.VMEM((PER_WORKER,), jnp.int32),
            pltpu.VMEM((NBUF, CHUNK, D_MODEL), jnp.float32),
            pltpu.SemaphoreType.DMA,
            pltpu.SemaphoreType.DMA,
        ],
        compiler_params=pltpu.CompilerParams(use_tc_tiling_on_sc=False),
    )
    def k(table_hbm, x_hbm, out_hbm, idx_v, bufs, gsem, wsem):
        num_cores = 2
        wid = lax.axis_index("s") * num_cores + lax.axis_index("c")
        pltpu.sync_copy(x_hbm.at[pl.ds(wid * PER_WORKER, PER_WORKER)], idx_v)
        out_base = wid * NCHUNK

        def gather(j, b):
            return pltpu.make_async_copy(
                table_hbm.at[idx_v.at[pl.ds(j * CHUNK, CHUNK)]],
                bufs.at[b], gsem)

        def wback(j, b):
            return pltpu.make_async_copy(
                bufs.at[b], out_hbm.at[out_base + j], wsem)

        for b in range(NBUF):
            gather(b, b).start()

        def group(g, carry):
            g0 = g * NBUF
            for b in range(NBUF):
                gather(g0 + b, b).wait()
                wback(g0 + b, b).start()
            for b in range(NBUF):
                wback(g0 + b, b).wait()
                gather(g0 + NBUF + b, b).start()
            return carry

        lax.fori_loop(0, NGROUPS - 1, group, 0)

        g0 = (NGROUPS - 1) * NBUF
        for b in range(NBUF):
            gather(g0 + b, b).wait()
            wback(g0 + b, b).start()
        for b in range(NBUF):
            wback(g0 + b, b).wait()

    return k


_gather_kernel = _make_kernel()


@jax.jit
def kernel(x, table):
    x_flat = x.reshape(B_TOTAL)
    out = _gather_kernel(table, x_flat)
    return out.reshape(x.shape[0], x.shape[1], D_MODEL)
